# baseline (speedup 1.0000x reference)
import functools

import jax
import jax.numpy as jnp
from jax import lax
from jax.experimental import pallas as pl
from jax.experimental.pallas import tpu as pltpu
from jax.experimental.pallas import tpu_sc as plsc

_B = 4096
_C = 1000
_NC = 2
_NS = 16
_L = 16
_NW = _NC * _NS
_ROWS_PER_W = _B // _NW
_NG = _ROWS_PER_W // _L

_CT = 40
_NCHUNK = _C // _CT

_R = 1.0
_DELTA = 0.5

_mesh = plsc.VectorSubcoreMesh(
    core_axis_name="c", subcore_axis_name="s",
    num_cores=_NC, num_subcores=_NS)


@functools.partial(
    pl.kernel,
    out_type=jax.ShapeDtypeStruct((_NW * _L,), jnp.float32),
    mesh=_mesh,
    scratch_types=[
        pltpu.VMEM((_CT, _ROWS_PER_W), jnp.float32),
        pltpu.VMEM((_CT, _ROWS_PER_W), jnp.float32),
        pltpu.VMEM((_ROWS_PER_W,), jnp.int32),
        pltpu.VMEM((_L,), jnp.float32),
        pltpu.SemaphoreType.DMA,
        pltpu.SemaphoreType.DMA,
    ],
    compiler_params=pltpu.CompilerParams(needs_layout_passes=False),
)
def _npt_loss_sc(dotT_hbm, tgt_hbm, out_hbm, buf0, buf1, tgt_v, out_v,
                 sem0, sem1):
    wid = lax.axis_index("s") * _NC + lax.axis_index("c")
    rbase = pl.multiple_of(wid * _ROWS_PER_W, _ROWS_PER_W)
    pltpu.sync_copy(tgt_hbm.at[pl.ds(rbase, _ROWS_PER_W)], tgt_v)

    bufs = (buf0, buf1)
    sems = (sem0, sem1)
    copies = [pltpu.async_copy(
        dotT_hbm.at[pl.ds(0, _CT), pl.ds(rbase, _ROWS_PER_W)], buf0, sem0),
        None]

    row_iota = lax.iota(jnp.int32, _L)
    zeros = jnp.zeros((_L,), jnp.float32)
    neg_inf = jnp.full((_L,), -jnp.inf, jnp.float32)

    tgts = [tgt_v[pl.ds(16 * l, _L)] for l in range(_NG)]
    cols = [row_iota + 16 * l for l in range(_NG)]
    m1 = [neg_inf] * _NG
    m2 = [neg_inf] * _NG
    tv = [zeros] * _NG

    for ch in range(_NCHUNK):
        buf = bufs[ch % 2]
        copies[ch % 2].wait()
        if ch + 1 < _NCHUNK:
            copies[(ch + 1) % 2] = pltpu.async_copy(
                dotT_hbm.at[pl.ds((ch + 1) * _CT, _CT),
                            pl.ds(rbase, _ROWS_PER_W)],
                bufs[(ch + 1) % 2], sems[(ch + 1) % 2])

        c0 = ch * _CT
        for l in range(_NG):
            inr = (tgts[l] >= c0) & (tgts[l] < c0 + _CT)
            idx = jnp.clip(tgts[l] - c0, 0, _CT - 1)
            got = plsc.load_gather(buf, [idx, cols[l]], mask=inr)
            tv[l] = jnp.where(inr, got, tv[l])
            plsc.store_scatter(buf, [idx, cols[l]], zeros, mask=inr)

        def body(tr, carry):
            cm1 = list(carry[:_NG])
            cm2 = list(carry[_NG:])
            rr = tr * 2
            for k in range(2):
                for l in range(_NG):
                    x = buf[rr + k, pl.ds(16 * l, _L)]
                    cm2[l] = jnp.maximum(cm2[l], jnp.minimum(cm1[l], x))
                    cm1[l] = jnp.maximum(cm1[l], x)
            return tuple(cm1) + tuple(cm2)

        carry = lax.fori_loop(0, _CT // 2, body, tuple(m1) + tuple(m2))
        m1 = list(carry[:_NG])
        m2 = list(carry[_NG:])

    acc = zeros
    for l in range(_NG):
        l1 = jnp.maximum(m1[l] - tv[l] + _DELTA, 0.0)
        l2 = jnp.maximum(m2[l] - tv[l] + _DELTA, 0.0)
        acc = acc + (l1 + l2)
    acc = acc * (2.0 * _R)

    out_v[...] = acc
    pltpu.sync_copy(out_v, out_hbm.at[pl.ds(wid * _L, _L)])


def kernel(dot_p, target):
    partials = _npt_loss_sc(dot_p.T, target.astype(jnp.int32))
    return jnp.sum(partials) / _B


# speedup vs baseline: 1.4661x; 1.4661x over previous
import functools

import jax
import jax.numpy as jnp
from jax import lax
from jax.experimental import pallas as pl
from jax.experimental.pallas import tpu as pltpu
from jax.experimental.pallas import tpu_sc as plsc

_B = 4096
_C = 1000
_NC = 2
_NS = 16
_L = 16
_NW = _NC * _NS
_ROWS_PER_W = _B // _NW
_NG = _ROWS_PER_W // _L

_CT = 200
_NCHUNK = _C // _CT

_R = 1.0
_DELTA = 0.5

_mesh = plsc.VectorSubcoreMesh(
    core_axis_name="c", subcore_axis_name="s",
    num_cores=_NC, num_subcores=_NS)


@functools.partial(
    pl.kernel,
    out_type=jax.ShapeDtypeStruct((_NW * _L,), jnp.float32),
    mesh=_mesh,
    scratch_types=[
        pltpu.VMEM((_CT, _ROWS_PER_W), jnp.float32),
        pltpu.VMEM((_CT, _ROWS_PER_W), jnp.float32),
        pltpu.VMEM((_ROWS_PER_W,), jnp.int32),
        pltpu.VMEM((_L,), jnp.float32),
        pltpu.SemaphoreType.DMA,
        pltpu.SemaphoreType.DMA,
    ],
    compiler_params=pltpu.CompilerParams(needs_layout_passes=False),
)
def _npt_loss_sc(dotT_hbm, tgt_hbm, out_hbm, buf0, buf1, tgt_v, out_v,
                 sem0, sem1):
    wid = lax.axis_index("s") * _NC + lax.axis_index("c")
    rbase = pl.multiple_of(wid * _ROWS_PER_W, _ROWS_PER_W)
    pltpu.sync_copy(tgt_hbm.at[pl.ds(rbase, _ROWS_PER_W)], tgt_v)

    bufs = (buf0, buf1)
    sems = (sem0, sem1)
    copies = [pltpu.async_copy(
        dotT_hbm.at[pl.ds(0, _CT), pl.ds(rbase, _ROWS_PER_W)], buf0, sem0),
        None]

    row_iota = lax.iota(jnp.int32, _L)
    zeros = jnp.zeros((_L,), jnp.float32)
    neg_inf = jnp.full((_L,), -jnp.inf, jnp.float32)

    tgts = [tgt_v[pl.ds(16 * l, _L)] for l in range(_NG)]
    cols = [row_iota + 16 * l for l in range(_NG)]
    m1 = [neg_inf] * _NG
    m2 = [neg_inf] * _NG
    tv = [zeros] * _NG

    for ch in range(_NCHUNK):
        buf = bufs[ch % 2]
        copies[ch % 2].wait()
        if ch + 1 < _NCHUNK:
            copies[(ch + 1) % 2] = pltpu.async_copy(
                dotT_hbm.at[pl.ds((ch + 1) * _CT, _CT),
                            pl.ds(rbase, _ROWS_PER_W)],
                bufs[(ch + 1) % 2], sems[(ch + 1) % 2])

        c0 = ch * _CT
        for l in range(_NG):
            inr = (tgts[l] >= c0) & (tgts[l] < c0 + _CT)
            idx = jnp.clip(tgts[l] - c0, 0, _CT - 1)
            got = plsc.load_gather(buf, [idx, cols[l]], mask=inr)
            tv[l] = jnp.where(inr, got, tv[l])
            plsc.store_scatter(buf, [idx, cols[l]], zeros, mask=inr)

        def body(tr, carry):
            cm1 = list(carry[:_NG])
            cm2 = list(carry[_NG:])
            rr = tr * 2
            for k in range(2):
                for l in range(_NG):
                    x = buf[rr + k, pl.ds(16 * l, _L)]
                    cm2[l] = jnp.maximum(cm2[l], jnp.minimum(cm1[l], x))
                    cm1[l] = jnp.maximum(cm1[l], x)
            return tuple(cm1) + tuple(cm2)

        carry = lax.fori_loop(0, _CT // 2, body, tuple(m1) + tuple(m2))
        m1 = list(carry[:_NG])
        m2 = list(carry[_NG:])

    acc = zeros
    for l in range(_NG):
        l1 = jnp.maximum(m1[l] - tv[l] + _DELTA, 0.0)
        l2 = jnp.maximum(m2[l] - tv[l] + _DELTA, 0.0)
        acc = acc + (l1 + l2)
    acc = acc * (2.0 * _R)

    out_v[...] = acc
    pltpu.sync_copy(out_v, out_hbm.at[pl.ds(wid * _L, _L)])


def kernel(dot_p, target):
    partials = _npt_loss_sc(dot_p.T, target.astype(jnp.int32))
    return jnp.sum(partials) / _B
